# baseline (speedup 1.0000x reference)
import jax
import jax.numpy as jnp
from jax import lax
from jax.experimental import pallas as pl
from jax.experimental.pallas import tpu as pltpu
from jax.experimental.pallas import tpu_sc as plsc

_VOL = (128, 128, 128)
_W = 12
_L = 16
_NC, _NS = 2, 16
_NW = _NC * _NS
_ROWS = _VOL[0] // _NW
_SLAB = _ROWS * _VOL[1] * _VOL[2]
_SCALE = 127.0


def _splat16(s, dtype=None):
    v = lax.broadcast_in_dim(s, (_L,), ())
    return v if dtype is None else v.astype(dtype)


def _sc_body(cx_h, cy_h, cz_h, sg_h, in_h, out_h,
             pcx, pcy, pcz, psg, pin, wl, acc):
    n = pcx.shape[0]
    w = lax.axis_index("s") * _NC + lax.axis_index("c")

    pltpu.sync_copy(cx_h, pcx)
    pltpu.sync_copy(cy_h, pcy)
    pltpu.sync_copy(cz_h, pcz)
    pltpu.sync_copy(sg_h, psg)
    pltpu.sync_copy(in_h, pin)

    zeros = jnp.zeros((_L,), jnp.float32)

    def zbody(i, carry):
        acc[pl.ds(i * _L, _L)] = zeros
        return carry

    lax.fori_loop(0, _SLAB // _L, zbody, 0)

    lane = lax.iota(jnp.int32, _L)
    lanef = lane.astype(jnp.float32)
    slab_lo = w * _ROWS
    slab_lo_f = _splat16(slab_lo, jnp.float32)
    slab_hi_f = slab_lo_f + float(_ROWS - 1)

    def gaussian(gs):
        cxs = plsc.load_gather(pcx, [gs])
        cys = plsc.load_gather(pcy, [gs])
        czs = plsc.load_gather(pcz, [gs])
        sgs = plsc.load_gather(psg, [gs])
        ins = plsc.load_gather(pin, [gs])
        cut = (3.0 * sgs) * _SCALE
        inv2 = 0.5 / (sgs * sgs)

        cvx = cxs * _SCALE
        cvy = cys * _SCALE
        cvz = czs * _SCALE
        lox = jnp.maximum(cvx - cut, 0.0).astype(jnp.int32).astype(jnp.float32)
        hix = jnp.minimum(cvx + cut, _SCALE).astype(jnp.int32).astype(jnp.float32)
        loy = jnp.maximum(cvy - cut, 0.0).astype(jnp.int32).astype(jnp.float32)
        hiy = jnp.minimum(cvy + cut, _SCALE).astype(jnp.int32).astype(jnp.float32)
        loz = jnp.maximum(cvz - cut, 0.0).astype(jnp.int32).astype(jnp.float32)
        hiz = jnp.minimum(cvz + cut, _SCALE).astype(jnp.int32).astype(jnp.float32)

        byf = jnp.clip(loy, 0.0, _SCALE - (_W - 1))
        bzf = jnp.clip(loz, 0.0, _SCALE - (_W - 1))
        by_i = byf.astype(jnp.int32)
        bz_i = bzf.astype(jnp.int32)

        zf = bzf + lanef
        dz = zf / _SCALE - czs
        wz = jnp.exp(-(dz * dz) * inv2)
        zmask = (zf >= loz) & (zf <= hiz)

        ninv2 = -inv2
        cs = []
        for yo in range(_W):
            yv = byf + float(yo)
            dy = yv / _SCALE - cys
            e = jnp.exp((dy * dy) * ninv2)
            m = (yv >= loy) & (yv <= hiy)
            cs.append(jnp.where(m, ins * e, 0.0))

        idx_base = by_i * _VOL[2] + bz_i + lane

        for xo in range(_ROWS):
            xfs = slab_lo_f + float(xo)
            xmask = (xfs >= lox) & (xfs <= hix)

            @pl.when(jnp.sum(xmask.astype(jnp.int32)) > 0)
            def _(xfs=xfs, xmask=xmask, xo=xo):
                dx = xfs / _SCALE - cxs
                wx = jnp.exp(-(dx * dx) * inv2)
                m = zmask & xmask
                row = xo * (_VOL[1] * _VOL[2])
                for yo in range(_W):
                    idx = idx_base + (row + yo * _VOL[2])
                    val = (wx * cs[yo]) * wz
                    plsc.addupdate_scatter(acc, [idx], val, mask=m)

    def scan_block(b, cnt):
        base = b * _L
        ids = base + lane
        cxv = pcx[pl.ds(base, _L)]
        sgv = psg[pl.ds(base, _L)]
        cvx = cxv * _SCALE
        cut = (3.0 * sgv) * _SCALE
        lox = jnp.maximum(cvx - cut, 0.0).astype(jnp.int32).astype(jnp.float32)
        hix = jnp.minimum(cvx + cut, _SCALE).astype(jnp.int32).astype(jnp.float32)
        hit = (hix >= slab_lo_f) & (lox <= slab_hi_f)
        hi32 = hit.astype(jnp.int32)
        pos = cnt + plsc.cumsum(hi32) - 1
        plsc.store_scatter(wl, [pos], ids, mask=hit)
        return cnt + jnp.sum(hi32)

    cnt = lax.fori_loop(0, n // _L, scan_block, 0)

    def work_block(bb, carry):
        @pl.when(bb * _L < cnt)
        def _():
            def inner(j, c2):
                i = bb * _L + j

                @pl.when(i < cnt)
                def _():
                    gaussian(plsc.load_gather(wl, [_splat16(i)]))

                return c2

            lax.fori_loop(0, _L, inner, 0)

        return carry

    lax.fori_loop(0, n // _L, work_block, 0)

    pltpu.sync_copy(acc, out_h.at[pl.ds(w * _SLAB, _SLAB)])


def kernel(centers, sigmas, intensities):
    n = centers.shape[0]
    pad = (-n) % _L
    if pad:
        centers = jnp.concatenate(
            [centers, jnp.full((pad, 3), 0.5, jnp.float32)], axis=0)
        sigmas = jnp.concatenate([sigmas, jnp.full((pad,), 0.004, jnp.float32)])
        intensities = jnp.concatenate(
            [intensities, jnp.zeros((pad,), jnp.float32)])
        n += pad
    cx = centers[:, 0]
    cy = centers[:, 1]
    cz = centers[:, 2]

    mesh = plsc.VectorSubcoreMesh(core_axis_name="c", subcore_axis_name="s")
    f = pl.kernel(
        _sc_body,
        out_type=jax.ShapeDtypeStruct((_VOL[0] * _VOL[1] * _VOL[2],),
                                      jnp.float32),
        mesh=mesh,
        compiler_params=pltpu.CompilerParams(needs_layout_passes=False),
        scratch_types=[
            pltpu.VMEM((n,), jnp.float32),
            pltpu.VMEM((n,), jnp.float32),
            pltpu.VMEM((n,), jnp.float32),
            pltpu.VMEM((n,), jnp.float32),
            pltpu.VMEM((n,), jnp.float32),
            pltpu.VMEM((n,), jnp.int32),
            pltpu.VMEM((_SLAB,), jnp.float32),
        ],
    )
    vol = f(cx, cy, cz, sigmas, intensities)
    return vol.reshape(_VOL)


# speedup vs baseline: 1.3244x; 1.3244x over previous
import jax
import jax.numpy as jnp
from jax import lax
from jax.experimental import pallas as pl
from jax.experimental.pallas import tpu as pltpu
from jax.experimental.pallas import tpu_sc as plsc

_VOL = (128, 128, 128)
_W = 12
_L = 16
_NC, _NS = 2, 16
_NW = _NC * _NS
_ROWS = _VOL[0] // _NW
_SLAB = _ROWS * _VOL[1] * _VOL[2]
_SCALE = 127.0


def _splat16(s, dtype=None):
    v = lax.broadcast_in_dim(s, (_L,), ())
    return v if dtype is None else v.astype(dtype)


def _sc_body(cx_h, cy_h, cz_h, sg_h, in_h, out_h,
             pcx, pcy, pcz, psg, pin, wl, acc):
    n = pcx.shape[0]
    w = lax.axis_index("s") * _NC + lax.axis_index("c")

    pltpu.sync_copy(cx_h, pcx)
    pltpu.sync_copy(cy_h, pcy)
    pltpu.sync_copy(cz_h, pcz)
    pltpu.sync_copy(sg_h, psg)
    pltpu.sync_copy(in_h, pin)

    zeros = jnp.zeros((_L,), jnp.float32)

    def zbody(i, carry):
        acc[pl.ds(i * _L, _L)] = zeros
        return carry

    lax.fori_loop(0, _SLAB // _L, zbody, 0)

    lane = lax.iota(jnp.int32, _L)
    lanef = lane.astype(jnp.float32)
    slab_lo = w * _ROWS
    slab_lo_f = _splat16(slab_lo, jnp.float32)
    slab_hi_f = slab_lo_f + float(_ROWS - 1)

    def gaussian(gs):
        cxs = plsc.load_gather(pcx, [gs])
        cys = plsc.load_gather(pcy, [gs])
        czs = plsc.load_gather(pcz, [gs])
        sgs = plsc.load_gather(psg, [gs])
        ins = plsc.load_gather(pin, [gs])
        cut = (3.0 * sgs) * _SCALE
        inv2 = 0.5 / (sgs * sgs)

        cvx = cxs * _SCALE
        cvy = cys * _SCALE
        cvz = czs * _SCALE
        lox = jnp.maximum(cvx - cut, 0.0).astype(jnp.int32).astype(jnp.float32)
        hix = jnp.minimum(cvx + cut, _SCALE).astype(jnp.int32).astype(jnp.float32)
        loy = jnp.maximum(cvy - cut, 0.0).astype(jnp.int32).astype(jnp.float32)
        hiy = jnp.minimum(cvy + cut, _SCALE).astype(jnp.int32).astype(jnp.float32)
        loz = jnp.maximum(cvz - cut, 0.0).astype(jnp.int32).astype(jnp.float32)
        hiz = jnp.minimum(cvz + cut, _SCALE).astype(jnp.int32).astype(jnp.float32)

        byf = jnp.clip(loy, 0.0, _SCALE - (_W - 1))
        bzf = jnp.clip(loz, 0.0, _SCALE - (_W - 1))
        by_i = byf.astype(jnp.int32)
        bz_i = bzf.astype(jnp.int32)

        zf = bzf + lanef
        dz = zf / _SCALE - czs
        wz = jnp.exp(-(dz * dz) * inv2)
        zmask = (zf >= loz) & (zf <= hiz)

        ninv2 = -inv2
        cs = []
        for yo in range(_W):
            yv = byf + float(yo)
            dy = yv / _SCALE - cys
            e = jnp.exp((dy * dy) * ninv2)
            m = (yv >= loy) & (yv <= hiy)
            cs.append(jnp.where(m, ins * e, 0.0))

        idx_base = by_i * _VOL[2] + bz_i + lane

        for xo in range(_ROWS):
            xfs = slab_lo_f + float(xo)
            dx = xfs / _SCALE - cxs
            wxz = jnp.exp(-(dx * dx) * inv2) * wz
            xmask = (xfs >= lox) & (xfs <= hix)
            m = zmask & xmask
            row = xo * (_VOL[1] * _VOL[2])
            for yo in range(_W):
                idx = idx_base + (row + yo * _VOL[2])
                plsc.addupdate_scatter(acc, [idx], wxz * cs[yo], mask=m)

    def scan_block(b, cnt):
        base = b * _L
        ids = base + lane
        cxv = pcx[pl.ds(base, _L)]
        sgv = psg[pl.ds(base, _L)]
        cvx = cxv * _SCALE
        cut = (3.0 * sgv) * _SCALE
        lox = jnp.maximum(cvx - cut, 0.0).astype(jnp.int32).astype(jnp.float32)
        hix = jnp.minimum(cvx + cut, _SCALE).astype(jnp.int32).astype(jnp.float32)
        hit = (hix >= slab_lo_f) & (lox <= slab_hi_f)
        hi32 = hit.astype(jnp.int32)
        pos = cnt + plsc.cumsum(hi32) - 1
        plsc.store_scatter(wl, [pos], ids, mask=hit)
        return cnt + jnp.sum(hi32)

    cnt = lax.fori_loop(0, n // _L, scan_block, 0)

    def work_block(bb, carry):
        @pl.when(bb * _L < cnt)
        def _():
            def inner(j, c2):
                i = bb * _L + j

                @pl.when(i < cnt)
                def _():
                    gaussian(plsc.load_gather(wl, [_splat16(i)]))

                return c2

            lax.fori_loop(0, _L, inner, 0)

        return carry

    lax.fori_loop(0, n // _L, work_block, 0)

    pltpu.sync_copy(acc, out_h.at[pl.ds(w * _SLAB, _SLAB)])


def kernel(centers, sigmas, intensities):
    n = centers.shape[0]
    pad = (-n) % _L
    if pad:
        centers = jnp.concatenate(
            [centers, jnp.full((pad, 3), 0.5, jnp.float32)], axis=0)
        sigmas = jnp.concatenate([sigmas, jnp.full((pad,), 0.004, jnp.float32)])
        intensities = jnp.concatenate(
            [intensities, jnp.zeros((pad,), jnp.float32)])
        n += pad
    cx = centers[:, 0]
    cy = centers[:, 1]
    cz = centers[:, 2]

    mesh = plsc.VectorSubcoreMesh(core_axis_name="c", subcore_axis_name="s")
    f = pl.kernel(
        _sc_body,
        out_type=jax.ShapeDtypeStruct((_VOL[0] * _VOL[1] * _VOL[2],),
                                      jnp.float32),
        mesh=mesh,
        compiler_params=pltpu.CompilerParams(needs_layout_passes=False),
        scratch_types=[
            pltpu.VMEM((n,), jnp.float32),
            pltpu.VMEM((n,), jnp.float32),
            pltpu.VMEM((n,), jnp.float32),
            pltpu.VMEM((n,), jnp.float32),
            pltpu.VMEM((n,), jnp.float32),
            pltpu.VMEM((n,), jnp.int32),
            pltpu.VMEM((_SLAB,), jnp.float32),
        ],
    )
    vol = f(cx, cy, cz, sigmas, intensities)
    return vol.reshape(_VOL)


# speedup vs baseline: 1.7261x; 1.3033x over previous
import jax
import jax.numpy as jnp
from jax import lax
from jax.experimental import pallas as pl
from jax.experimental.pallas import tpu as pltpu
from jax.experimental.pallas import tpu_sc as plsc

_VOL = (128, 128, 128)
_W = 12
_L = 16
_NC, _NS = 2, 16
_NW = _NC * _NS
_ROWS = _VOL[0] // _NW
_SLAB = _ROWS * _VOL[1] * _VOL[2]
_SCALE = 127.0


def _splat16(s, dtype=None):
    v = lax.broadcast_in_dim(s, (_L,), ())
    return v if dtype is None else v.astype(dtype)


def _sc_body(cx_h, cy_h, cz_h, sg_h, in_h, out_h,
             pcx, pcy, pcz, psg, pin, wl, acc):
    n = pcx.shape[0]
    w = lax.axis_index("s") * _NC + lax.axis_index("c")

    pltpu.sync_copy(cx_h, pcx)
    pltpu.sync_copy(cy_h, pcy)
    pltpu.sync_copy(cz_h, pcz)
    pltpu.sync_copy(sg_h, psg)
    pltpu.sync_copy(in_h, pin)

    zeros = jnp.zeros((_L,), jnp.float32)

    def zbody(i, carry):
        base = i * (8 * _L)
        for u in range(8):
            acc[pl.ds(base + u * _L, _L)] = zeros
        return carry

    lax.fori_loop(0, _SLAB // (8 * _L), zbody, 0)

    lane = lax.iota(jnp.int32, _L)
    lanef = lane.astype(jnp.float32)
    slab_lo = w * _ROWS
    slab_lo_f = _splat16(slab_lo, jnp.float32)
    slab_hi_f = slab_lo_f + float(_ROWS - 1)

    def gaussian(gs):
        cxs = plsc.load_gather(pcx, [gs])
        cys = plsc.load_gather(pcy, [gs])
        czs = plsc.load_gather(pcz, [gs])
        sgs = plsc.load_gather(psg, [gs])
        ins = plsc.load_gather(pin, [gs])
        cut = (3.0 * sgs) * _SCALE
        inv2 = 0.5 / (sgs * sgs)
        ninv2 = -inv2

        cvx = cxs * _SCALE
        cvy = cys * _SCALE
        cvz = czs * _SCALE
        lox = jnp.maximum(cvx - cut, 0.0).astype(jnp.int32).astype(jnp.float32)
        hix = jnp.minimum(cvx + cut, _SCALE).astype(jnp.int32).astype(jnp.float32)
        loy = jnp.maximum(cvy - cut, 0.0).astype(jnp.int32).astype(jnp.float32)
        hiy = jnp.minimum(cvy + cut, _SCALE).astype(jnp.int32).astype(jnp.float32)
        loz = jnp.maximum(cvz - cut, 0.0).astype(jnp.int32).astype(jnp.float32)
        hiz = jnp.minimum(cvz + cut, _SCALE).astype(jnp.int32).astype(jnp.float32)

        byf = jnp.minimum(loy, _SCALE - (_W - 1))
        bzf = jnp.minimum(loz, _SCALE - (_W - 1))

        zf = bzf + lanef
        dz = zf * (1.0 / _SCALE) - czs
        wz = jnp.exp(-(dz * dz) * inv2) * ins
        zmask = (zf >= loz) & (zf <= hiz)

        dy0 = byf * (1.0 / _SCALE) - cys
        ay = loy - byf
        by2 = hiy - byf
        cs = []
        for yo in range(_W):
            dy = dy0 + float(yo) / _SCALE
            e = jnp.exp((dy * dy) * ninv2)
            m = (float(yo) >= ay) & (float(yo) <= by2)
            cs.append(jnp.where(m, e, 0.0))

        idx_base = (byf * _VOL[2] + bzf).astype(jnp.int32) + lane
        idxs = [idx_base + yo * _VOL[2] for yo in range(_W)]

        for xo in range(_ROWS):
            xfs = slab_lo_f + float(xo)
            dx = xfs * (1.0 / _SCALE) - cxs
            wxz = jnp.exp(-(dx * dx) * inv2) * wz
            xmask = (xfs >= lox) & (xfs <= hix)
            m = zmask & xmask
            rowref = acc.at[pl.ds(xo * (_VOL[1] * _VOL[2]),
                                  _VOL[1] * _VOL[2])]
            for yo in range(_W):
                plsc.addupdate_scatter(rowref, [idxs[yo]], wxz * cs[yo],
                                       mask=m)

    def scan_block(b, cnt):
        base = b * _L
        ids = base + lane
        cxv = pcx[pl.ds(base, _L)]
        sgv = psg[pl.ds(base, _L)]
        cvx = cxv * _SCALE
        cut = (3.0 * sgv) * _SCALE
        lox = jnp.maximum(cvx - cut, 0.0).astype(jnp.int32).astype(jnp.float32)
        hix = jnp.minimum(cvx + cut, _SCALE).astype(jnp.int32).astype(jnp.float32)
        hit = (hix >= slab_lo_f) & (lox <= slab_hi_f)
        hi32 = hit.astype(jnp.int32)
        pos = cnt + plsc.cumsum(hi32) - 1
        plsc.store_scatter(wl, [pos], ids, mask=hit)
        return cnt + jnp.sum(hi32)

    cnt = lax.fori_loop(0, n // _L, scan_block, 0)

    def work_block(bb, carry):
        @pl.when(bb * _L < cnt)
        def _():
            def inner(j, c2):
                i = bb * _L + j

                @pl.when(i < cnt)
                def _():
                    gaussian(plsc.load_gather(wl, [_splat16(i)]))

                return c2

            lax.fori_loop(0, _L, inner, 0)

        return carry

    lax.fori_loop(0, n // _L, work_block, 0)

    pltpu.sync_copy(acc, out_h.at[pl.ds(w * _SLAB, _SLAB)])


def kernel(centers, sigmas, intensities):
    n = centers.shape[0]
    pad = (-n) % _L
    if pad:
        centers = jnp.concatenate(
            [centers, jnp.full((pad, 3), 0.5, jnp.float32)], axis=0)
        sigmas = jnp.concatenate([sigmas, jnp.full((pad,), 0.004, jnp.float32)])
        intensities = jnp.concatenate(
            [intensities, jnp.zeros((pad,), jnp.float32)])
        n += pad
    cx = centers[:, 0]
    cy = centers[:, 1]
    cz = centers[:, 2]

    mesh = plsc.VectorSubcoreMesh(core_axis_name="c", subcore_axis_name="s")
    f = pl.kernel(
        _sc_body,
        out_type=jax.ShapeDtypeStruct((_VOL[0] * _VOL[1] * _VOL[2],),
                                      jnp.float32),
        mesh=mesh,
        compiler_params=pltpu.CompilerParams(needs_layout_passes=False),
        scratch_types=[
            pltpu.VMEM((n,), jnp.float32),
            pltpu.VMEM((n,), jnp.float32),
            pltpu.VMEM((n,), jnp.float32),
            pltpu.VMEM((n,), jnp.float32),
            pltpu.VMEM((n,), jnp.float32),
            pltpu.VMEM((n,), jnp.int32),
            pltpu.VMEM((_SLAB,), jnp.float32),
        ],
    )
    vol = f(cx, cy, cz, sigmas, intensities)
    return vol.reshape(_VOL)


# speedup vs baseline: 1.7681x; 1.0243x over previous
import jax
import jax.numpy as jnp
from jax import lax
from jax.experimental import pallas as pl
from jax.experimental.pallas import tpu as pltpu
from jax.experimental.pallas import tpu_sc as plsc

_VOL = (128, 128, 128)
_W = 12
_WY = 11
_L = 16
_NC, _NS = 2, 16
_NW = _NC * _NS
_ROWS = _VOL[0] // _NW
_SLAB = _ROWS * _VOL[1] * _VOL[2]
_SCALE = 127.0


def _splat16(s, dtype=None):
    v = lax.broadcast_in_dim(s, (_L,), ())
    return v if dtype is None else v.astype(dtype)


def _sc_body(cx_h, cy_h, cz_h, sg_h, in_h, out_h,
             pcx, pcy, pcz, psg, pin, wl, acc):
    n = pcx.shape[0]
    w = lax.axis_index("s") * _NC + lax.axis_index("c")

    pltpu.sync_copy(cx_h, pcx)
    pltpu.sync_copy(cy_h, pcy)
    pltpu.sync_copy(cz_h, pcz)
    pltpu.sync_copy(sg_h, psg)
    pltpu.sync_copy(in_h, pin)

    zeros = jnp.zeros((_L,), jnp.float32)

    def zbody(i, carry):
        base = i * (8 * _L)
        for u in range(8):
            acc[pl.ds(base + u * _L, _L)] = zeros
        return carry

    lax.fori_loop(0, _SLAB // (8 * _L), zbody, 0)

    lane = lax.iota(jnp.int32, _L)
    lanef = lane.astype(jnp.float32)
    slab_lo = w * _ROWS
    slab_lo_f = _splat16(slab_lo, jnp.float32)
    slab_hi_f = slab_lo_f + float(_ROWS - 1)

    def gaussian(gs):
        cxs = plsc.load_gather(pcx, [gs])
        cys = plsc.load_gather(pcy, [gs])
        czs = plsc.load_gather(pcz, [gs])
        sgs = plsc.load_gather(psg, [gs])
        ins = plsc.load_gather(pin, [gs])
        cut = (3.0 * sgs) * _SCALE
        inv2 = 0.5 / (sgs * sgs)
        ninv2 = -inv2

        cvx = cxs * _SCALE
        cvy = cys * _SCALE
        cvz = czs * _SCALE
        lox = jnp.maximum(cvx - cut, 0.0).astype(jnp.int32).astype(jnp.float32)
        hix = jnp.minimum(cvx + cut, _SCALE).astype(jnp.int32).astype(jnp.float32)
        loy = jnp.maximum(cvy - cut, 0.0).astype(jnp.int32).astype(jnp.float32)
        hiy = jnp.minimum(cvy + cut, _SCALE).astype(jnp.int32).astype(jnp.float32)
        loz = jnp.maximum(cvz - cut, 0.0).astype(jnp.int32).astype(jnp.float32)
        hiz = jnp.minimum(cvz + cut, _SCALE).astype(jnp.int32).astype(jnp.float32)

        byf = jnp.minimum(loy, _SCALE - (_WY - 1))
        bzf = jnp.minimum(loz, _SCALE - (_WY - 1))

        zf = bzf + lanef
        dz = zf * (1.0 / _SCALE) - czs
        wz = jnp.exp(-(dz * dz) * inv2) * ins
        zmask = (zf >= loz) & (zf <= hiz)

        dy0 = byf * (1.0 / _SCALE) - cys
        ay = loy - byf
        by2 = hiy - byf
        cs = []
        for yo in range(_WY):
            dy = dy0 + float(yo) / _SCALE
            e = jnp.exp((dy * dy) * ninv2)
            m = (float(yo) >= ay) & (float(yo) <= by2)
            cs.append(jnp.where(m, e, 0.0))

        idx_base = (byf * _VOL[2] + bzf).astype(jnp.int32) + lane
        idxs = [idx_base + yo * _VOL[2] for yo in range(_WY)]

        for xo in range(_ROWS):
            xfs = slab_lo_f + float(xo)
            dx = xfs * (1.0 / _SCALE) - cxs
            wxz = jnp.exp(-(dx * dx) * inv2) * wz
            xmask = (xfs >= lox) & (xfs <= hix)
            m = zmask & xmask
            rowref = acc.at[pl.ds(xo * (_VOL[1] * _VOL[2]),
                                  _VOL[1] * _VOL[2])]
            for yo in range(_WY):
                plsc.addupdate_scatter(rowref, [idxs[yo]], wxz * cs[yo],
                                       mask=m)

    def scan_block(b, cnt):
        base = b * _L
        ids = base + lane
        cxv = pcx[pl.ds(base, _L)]
        sgv = psg[pl.ds(base, _L)]
        cvx = cxv * _SCALE
        cut = (3.0 * sgv) * _SCALE
        lox = jnp.maximum(cvx - cut, 0.0).astype(jnp.int32).astype(jnp.float32)
        hix = jnp.minimum(cvx + cut, _SCALE).astype(jnp.int32).astype(jnp.float32)
        hit = (hix >= slab_lo_f) & (lox <= slab_hi_f)
        hi32 = hit.astype(jnp.int32)
        pos = cnt + plsc.cumsum(hi32) - 1
        plsc.store_scatter(wl, [pos], ids, mask=hit)
        return cnt + jnp.sum(hi32)

    cnt = lax.fori_loop(0, n // _L, scan_block, 0)

    def work_block(bb, carry):
        @pl.when(bb * _L < cnt)
        def _():
            def inner(j, c2):
                i = bb * _L + j

                @pl.when(i < cnt)
                def _():
                    gaussian(plsc.load_gather(wl, [_splat16(i)]))

                return c2

            lax.fori_loop(0, _L, inner, 0)

        return carry

    lax.fori_loop(0, n // _L, work_block, 0)

    pltpu.sync_copy(acc, out_h.at[pl.ds(w * _SLAB, _SLAB)])


def kernel(centers, sigmas, intensities):
    n = centers.shape[0]
    pad = (-n) % _L
    if pad:
        centers = jnp.concatenate(
            [centers, jnp.full((pad, 3), 0.5, jnp.float32)], axis=0)
        sigmas = jnp.concatenate([sigmas, jnp.full((pad,), 0.004, jnp.float32)])
        intensities = jnp.concatenate(
            [intensities, jnp.zeros((pad,), jnp.float32)])
        n += pad
    cx = centers[:, 0]
    cy = centers[:, 1]
    cz = centers[:, 2]

    mesh = plsc.VectorSubcoreMesh(core_axis_name="c", subcore_axis_name="s")
    f = pl.kernel(
        _sc_body,
        out_type=jax.ShapeDtypeStruct((_VOL[0] * _VOL[1] * _VOL[2],),
                                      jnp.float32),
        mesh=mesh,
        compiler_params=pltpu.CompilerParams(needs_layout_passes=False),
        scratch_types=[
            pltpu.VMEM((n,), jnp.float32),
            pltpu.VMEM((n,), jnp.float32),
            pltpu.VMEM((n,), jnp.float32),
            pltpu.VMEM((n,), jnp.float32),
            pltpu.VMEM((n,), jnp.float32),
            pltpu.VMEM((n,), jnp.int32),
            pltpu.VMEM((_SLAB,), jnp.float32),
        ],
    )
    vol = f(cx, cy, cz, sigmas, intensities)
    return vol.reshape(_VOL)


# speedup vs baseline: 1.8046x; 1.0206x over previous
import jax
import jax.numpy as jnp
from jax import lax
from jax.experimental import pallas as pl
from jax.experimental.pallas import tpu as pltpu
from jax.experimental.pallas import tpu_sc as plsc

_VOL = (128, 128, 128)
_W = 12
_WY = 11
_L = 16
_NC, _NS = 2, 16
_NW = _NC * _NS
_ROWS = _VOL[0] // _NW
_SLAB = _ROWS * _VOL[1] * _VOL[2]
_SCALE = 127.0


def _splat16(s, dtype=None):
    v = lax.broadcast_in_dim(s, (_L,), ())
    return v if dtype is None else v.astype(dtype)


def _sc_body(cx_h, cy_h, cz_h, sg_h, in_h, out_h,
             pcx, pcy, pcz, psg, pin, wl, acc):
    n = pcx.shape[0]
    w = lax.axis_index("s") * _NC + lax.axis_index("c")

    pltpu.sync_copy(cx_h, pcx)
    pltpu.sync_copy(cy_h, pcy)
    pltpu.sync_copy(cz_h, pcz)
    pltpu.sync_copy(sg_h, psg)
    pltpu.sync_copy(in_h, pin)

    zeros = jnp.zeros((_L,), jnp.float32)

    def zbody(i, carry):
        base = i * (8 * _L)
        for u in range(8):
            acc[pl.ds(base + u * _L, _L)] = zeros
        return carry

    lax.fori_loop(0, _SLAB // (8 * _L), zbody, 0)

    lane = lax.iota(jnp.int32, _L)
    lanef = lane.astype(jnp.float32)
    slab_lo = w * _ROWS
    slab_lo_f = _splat16(slab_lo, jnp.float32)
    slab_hi_f = slab_lo_f + float(_ROWS - 1)

    def gaussian(gs):
        cxs = plsc.load_gather(pcx, [gs])
        cys = plsc.load_gather(pcy, [gs])
        czs = plsc.load_gather(pcz, [gs])
        sgs = plsc.load_gather(psg, [gs])
        ins = plsc.load_gather(pin, [gs])
        cut = (3.0 * sgs) * _SCALE
        inv2 = 0.5 / (sgs * sgs)
        ninv2 = -inv2

        cvx = cxs * _SCALE
        cvy = cys * _SCALE
        cvz = czs * _SCALE
        lox = jnp.maximum(cvx - cut, 0.0).astype(jnp.int32).astype(jnp.float32)
        hix = jnp.minimum(cvx + cut, _SCALE).astype(jnp.int32).astype(jnp.float32)
        loy = jnp.maximum(cvy - cut, 0.0).astype(jnp.int32).astype(jnp.float32)
        hiy = jnp.minimum(cvy + cut, _SCALE).astype(jnp.int32).astype(jnp.float32)
        loz = jnp.maximum(cvz - cut, 0.0).astype(jnp.int32).astype(jnp.float32)
        hiz = jnp.minimum(cvz + cut, _SCALE).astype(jnp.int32).astype(jnp.float32)

        byf = jnp.minimum(loy, _SCALE - (_WY - 1))
        bzf = jnp.minimum(loz, _SCALE - (_WY - 1))

        zf = bzf + lanef
        dz = zf * (1.0 / _SCALE) - czs
        wz = jnp.exp(-(dz * dz) * inv2) * ins
        zmask = (zf >= loz) & (zf <= hiz)

        dy0 = byf * (1.0 / _SCALE) - cys
        ay = loy - byf
        by2 = hiy - byf
        cs = []
        for yo in range(_WY):
            dy = dy0 + float(yo) / _SCALE
            e = jnp.exp((dy * dy) * ninv2)
            m = (float(yo) >= ay) & (float(yo) <= by2)
            cs.append(jnp.where(m, e, 0.0))

        idx_base = (byf * _VOL[2] + bzf).astype(jnp.int32) + lane
        idxs = [idx_base + yo * _VOL[2] for yo in range(_WY)]

        for xo in range(_ROWS):
            xfs = slab_lo_f + float(xo)
            dx = xfs * (1.0 / _SCALE) - cxs
            wxz = jnp.exp(-(dx * dx) * inv2) * wz
            xmask = (xfs >= lox) & (xfs <= hix)
            m = zmask & xmask
            rowref = acc.at[pl.ds(xo * (_VOL[1] * _VOL[2]),
                                  _VOL[1] * _VOL[2])]
            for yo in range(_WY):
                plsc.addupdate_scatter(rowref, [idxs[yo]], wxz * cs[yo],
                                       mask=m)

    def scan_block(b, cnt):
        base = b * _L
        ids = base + lane
        cxv = pcx[pl.ds(base, _L)]
        sgv = psg[pl.ds(base, _L)]
        cvx = cxv * _SCALE
        cut = (3.0 * sgv) * _SCALE
        lox = jnp.maximum(cvx - cut, 0.0).astype(jnp.int32).astype(jnp.float32)
        hix = jnp.minimum(cvx + cut, _SCALE).astype(jnp.int32).astype(jnp.float32)
        hit = (hix >= slab_lo_f) & (lox <= slab_hi_f)
        hi32 = hit.astype(jnp.int32)
        pos = cnt + plsc.cumsum(hi32) - 1
        plsc.store_scatter(wl, [pos], ids, mask=hit)
        return cnt + jnp.sum(hi32)

    cnt = lax.fori_loop(0, n // _L, scan_block, 0)

    plsc.store_scatter(wl, [_splat16(cnt)], _splat16(n - 1), mask=lane < 1)

    def work_block(bb, carry):
        @pl.when(bb * _L < cnt)
        def _():
            def inner(j, c2):
                i = bb * _L + 2 * j

                @pl.when(i < cnt)
                def _():
                    gaussian(plsc.load_gather(wl, [_splat16(i)]))
                    gaussian(plsc.load_gather(wl, [_splat16(i + 1)]))

                return c2

            lax.fori_loop(0, _L // 2, inner, 0)

        return carry

    lax.fori_loop(0, n // _L, work_block, 0)

    pltpu.sync_copy(acc, out_h.at[pl.ds(w * _SLAB, _SLAB)])


def kernel(centers, sigmas, intensities):
    n = centers.shape[0]
    pad = (-n) % _L + _L
    centers = jnp.concatenate(
        [centers, jnp.full((pad, 3), 0.5, jnp.float32)], axis=0)
    sigmas = jnp.concatenate([sigmas, jnp.full((pad,), 0.004, jnp.float32)])
    intensities = jnp.concatenate(
        [intensities, jnp.zeros((pad,), jnp.float32)])
    n += pad
    cx = centers[:, 0]
    cy = centers[:, 1]
    cz = centers[:, 2]

    mesh = plsc.VectorSubcoreMesh(core_axis_name="c", subcore_axis_name="s")
    f = pl.kernel(
        _sc_body,
        out_type=jax.ShapeDtypeStruct((_VOL[0] * _VOL[1] * _VOL[2],),
                                      jnp.float32),
        mesh=mesh,
        compiler_params=pltpu.CompilerParams(needs_layout_passes=False),
        scratch_types=[
            pltpu.VMEM((n,), jnp.float32),
            pltpu.VMEM((n,), jnp.float32),
            pltpu.VMEM((n,), jnp.float32),
            pltpu.VMEM((n,), jnp.float32),
            pltpu.VMEM((n,), jnp.float32),
            pltpu.VMEM((n + _L,), jnp.int32),
            pltpu.VMEM((_SLAB,), jnp.float32),
        ],
    )
    vol = f(cx, cy, cz, sigmas, intensities)
    return vol.reshape(_VOL)
